# baseline (speedup 1.0000x reference)
import functools

import jax
import jax.numpy as jnp
from jax import lax
from jax.experimental import pallas as pl
from jax.experimental.pallas import tpu as pltpu
from jax.experimental.pallas import tpu_sc as plsc

_VOCAB = 1000000
_SEQ = 200
_D = 64
_BATCH = 4096

_NC = 2
_NS = 16
_NW = _NC * _NS

_ROWS = _BATCH * _SEQ
_ROWS_PER_W = _ROWS // _NW
_SEQS_PER_W = _ROWS_PER_W // _SEQ
_LANES = 16
_SLICES_PER_ROW = _D // _LANES


def _body(table_hbm, idx_hbm, pe_hbm, out_hbm, idx_v, buf_v, pe_v, sem):
    wid = lax.axis_index("s") * _NC + lax.axis_index("c")
    base = wid * _ROWS_PER_W

    pltpu.sync_copy(pe_hbm, pe_v)

    def chunk_body(c, carry):
        row0 = base + c * _SEQ
        pltpu.sync_copy(idx_hbm.at[pl.ds(row0, _SEQ)], idx_v)
        pltpu.async_copy(table_hbm.at[idx_v], buf_v, sem).wait()

        def add_row(r, carry2):
            for j in range(_SLICES_PER_ROW):
                sl = pl.ds(j * _LANES, _LANES)
                buf_v[r, sl] = buf_v[r, sl] + pe_v[r, sl]
            return carry2

        lax.fori_loop(0, _SEQ, add_row, 0)
        pltpu.sync_copy(buf_v, out_hbm.at[pl.ds(row0, _SEQ)])
        return carry

    lax.fori_loop(0, _SEQS_PER_W, chunk_body, 0)


@jax.jit
def _sc_embed(table, idx, pe):
    mesh = plsc.VectorSubcoreMesh(core_axis_name="c", subcore_axis_name="s")
    return pl.kernel(
        _body,
        out_type=jax.ShapeDtypeStruct((_ROWS, _D), jnp.float32),
        mesh=mesh,
        scratch_types=[
            pltpu.VMEM((_SEQ,), jnp.int32),
            pltpu.VMEM((_SEQ, _D), jnp.float32),
            pltpu.VMEM((_SEQ, _D), jnp.float32),
            pltpu.SemaphoreType.DMA,
        ],
        compiler_params=pltpu.CompilerParams(use_tc_tiling_on_sc=False),
    )(table, idx, pe)


def kernel(indices, table, pos_enc):
    idx = indices.reshape(-1).astype(jnp.int32)
    out = _sc_embed(table, idx, pos_enc)
    return out.reshape(_BATCH, _SEQ, _D)


# speedup vs baseline: 1.1589x; 1.1589x over previous
import jax
import jax.numpy as jnp
from jax import lax
from jax.experimental import pallas as pl
from jax.experimental.pallas import tpu as pltpu
from jax.experimental.pallas import tpu_sc as plsc

_VOCAB = 1000000
_SEQ = 200
_D = 64
_BATCH = 4096

_NC = 2
_NS = 16
_NW = _NC * _NS

_SEQS_PER_W = _BATCH // _NW
_NBUF = 4
_NOUTER = _SEQS_PER_W // _NBUF
_LANES = 16
_SLICES = _D // _LANES


def _body(table_hbm, idx_hbm, pe_hbm, out_hbm,
          idx_all, pe_v, bufs, gsems, osems):
    wid = lax.axis_index("s") * _NC + lax.axis_index("c")
    seq0 = wid * _SEQS_PER_W

    pltpu.sync_copy(idx_hbm.at[pl.ds(seq0, _SEQS_PER_W)], idx_all)
    pltpu.sync_copy(pe_hbm, pe_v)

    def _gather(g, b):
        pltpu.async_copy(table_hbm.at[idx_all.at[g]], bufs[b], gsems[b])

    def _wait_gather(g, b):
        pltpu.make_async_copy(table_hbm.at[idx_all.at[g]], bufs[b],
                              gsems[b]).wait()

    def _out_slice(g):
        return out_hbm.at[pl.ds((seq0 + g) * _SEQ, _SEQ)]

    for b in range(_NBUF):
        _gather(b, b)

    def outer(o, carry):
        g0 = o * _NBUF
        for b in range(_NBUF):
            g = g0 + b
            _wait_gather(g, b)
            buf = bufs[b]

            @plsc.parallel_loop(0, _SEQ, 1, unroll=2)
            def add_row(r):
                for j in range(_SLICES):
                    sl = pl.ds(j * _LANES, _LANES)
                    buf[r, sl] = buf[r, sl] + pe_v[r, sl]

            pltpu.async_copy(buf, _out_slice(g), osems[b])
        for b in range(_NBUF):
            g = g0 + b

            @pl.when(o < _NOUTER - 1)
            def _():
                pltpu.make_async_copy(bufs[b], _out_slice(g), osems[b]).wait()
                _gather(g + _NBUF, b)

        return carry

    lax.fori_loop(0, _NOUTER, outer, 0)

    for b in range(_NBUF):
        g = (_NOUTER - 1) * _NBUF + b
        pltpu.make_async_copy(bufs[b], _out_slice(g), osems[b]).wait()


def _kernel_body(table_hbm, idx_hbm, pe_hbm, out_hbm,
                 idx_all, pe_v, b0, b1, b2, b3,
                 g0, g1, g2, g3, o0, o1, o2, o3):
    _body(table_hbm, idx_hbm, pe_hbm, out_hbm, idx_all, pe_v,
          [b0, b1, b2, b3], [g0, g1, g2, g3], [o0, o1, o2, o3])


@jax.jit
def _sc_embed(table, idx, pe):
    mesh = plsc.VectorSubcoreMesh(core_axis_name="c", subcore_axis_name="s")
    return pl.kernel(
        _kernel_body,
        out_type=jax.ShapeDtypeStruct((_BATCH * _SEQ, _D), jnp.float32),
        mesh=mesh,
        scratch_types=(
            [pltpu.VMEM((_SEQS_PER_W, _SEQ), jnp.int32),
             pltpu.VMEM((_SEQ, _D), jnp.float32)]
            + [pltpu.VMEM((_SEQ, _D), jnp.float32) for _ in range(_NBUF)]
            + [pltpu.SemaphoreType.DMA for _ in range(2 * _NBUF)]
        ),
        compiler_params=pltpu.CompilerParams(use_tc_tiling_on_sc=False),
    )(table, idx, pe)


def kernel(indices, table, pos_enc):
    idx = indices.astype(jnp.int32)
    out = _sc_embed(table, idx, pos_enc)
    return out.reshape(_BATCH, _SEQ, _D)
